# Initial kernel scaffold; baseline (speedup 1.0000x reference)
import jax
import jax.numpy as jnp
from jax.experimental import pallas as pl


def kernel(x, xyz, W1, b1, W2, b2):
    raise NotImplementedError("write your pallas kernel here")


# speedup vs baseline: 16.9054x; 16.9054x over previous
import functools

import jax
import jax.numpy as jnp
from jax import lax
from jax.experimental import pallas as pl
from jax.experimental.pallas import tpu as pltpu
from jax.experimental.pallas import tpu_sc as plsc

_B, _C, _N = 8, 128, 4096
_M = 1024
_K = 16


def _fps_body(x_ref, y_ref, z_ref, idx_ref, sx_ref, sy_ref, sz_ref, dists_ref):
    X = x_ref[...]
    Y = y_ref[...]
    Z = z_ref[...]
    iota_n = lax.broadcasted_iota(jnp.int32, (_B, _N), 1)
    lane_m = lax.broadcasted_iota(jnp.int32, (_B, _M), 1)
    dists_ref[...] = jnp.full((_B, _N), 1e10, dtype=jnp.float32)
    idx_ref[...] = jnp.zeros((_B, _M), jnp.int32)
    sx_ref[...] = jnp.zeros((_B, _M), jnp.float32)
    sy_ref[...] = jnp.zeros((_B, _M), jnp.float32)
    sz_ref[...] = jnp.zeros((_B, _M), jnp.float32)

    def body(i, far):
        hit = lane_m == i
        idx_ref[...] = jnp.where(hit, far, idx_ref[...])
        sel = iota_n == far
        zero = jnp.float32(0.0)
        cx = jnp.sum(jnp.where(sel, X, zero), axis=1, keepdims=True)
        cy = jnp.sum(jnp.where(sel, Y, zero), axis=1, keepdims=True)
        cz = jnp.sum(jnp.where(sel, Z, zero), axis=1, keepdims=True)
        sx_ref[...] = jnp.where(hit, cx, sx_ref[...])
        sy_ref[...] = jnp.where(hit, cy, sy_ref[...])
        sz_ref[...] = jnp.where(hit, cz, sz_ref[...])
        dx = X - cx
        dy = Y - cy
        dz = Z - cz
        d = dx * dx + dy * dy + dz * dz
        dn = jnp.minimum(dists_ref[...], d)
        dists_ref[...] = dn
        m = jnp.max(dn, axis=1, keepdims=True)
        cand = jnp.where(dn == m, iota_n, _N)
        return jnp.min(cand, axis=1, keepdims=True).astype(jnp.int32)

    lax.fori_loop(0, _M, body, jnp.zeros((_B, 1), jnp.int32))


def _fps(xp, yp, zp, interpret=False):
    return pl.pallas_call(
        _fps_body,
        out_shape=[
            jax.ShapeDtypeStruct((_B, _M), jnp.int32),
            jax.ShapeDtypeStruct((_B, _M), jnp.float32),
            jax.ShapeDtypeStruct((_B, _M), jnp.float32),
            jax.ShapeDtypeStruct((_B, _M), jnp.float32),
        ],
        scratch_shapes=[pltpu.VMEM((_B, _N), jnp.float32)],
        interpret=interpret,
    )(xp, yp, zp)


def _sc_gather(table, gidx, rows_per_chunk):
    (R,) = gidx.shape
    V, D = table.shape
    info = plsc.get_sparse_core_info()
    nw = info.num_cores * info.num_subcores
    per_w = R // nw
    n_chunks = per_w // rows_per_chunk
    mesh = plsc.VectorSubcoreMesh(core_axis_name="c", subcore_axis_name="s")

    @functools.partial(
        pl.kernel,
        mesh=mesh,
        out_type=jax.ShapeDtypeStruct((R, D), jnp.float32),
        scratch_types=[
            pltpu.VMEM((rows_per_chunk,), jnp.int32),
            pltpu.VMEM((rows_per_chunk, D), jnp.float32),
            pltpu.SemaphoreType.DMA,
        ],
    )
    def gk(table_hbm, idx_hbm, out_hbm, idx_v, rows_v, sem):
        wid = lax.axis_index("s") * info.num_cores + lax.axis_index("c")

        def body(ci, carry):
            base = wid * per_w + ci * rows_per_chunk
            pltpu.sync_copy(idx_hbm.at[pl.ds(base, rows_per_chunk)], idx_v)
            pltpu.async_copy(table_hbm.at[idx_v], rows_v, sem).wait()
            pltpu.sync_copy(rows_v, out_hbm.at[pl.ds(base, rows_per_chunk)])
            return carry

        lax.fori_loop(0, n_chunks, body, 0)

    return gk(table, gidx)


_MT = 256


def _topk_body(q_ref, xt_ref, x_ref, out_ref):
    q = q_ref[0]
    xt = xt_ref[0]
    xsq = x_ref[0]
    qq = jnp.sum(q * q, axis=1, keepdims=True)
    xx = jnp.sum(xsq * xsq, axis=0, keepdims=True)
    d2 = lax.dot_general(q, xt, (((1,), (1,)), ((), ())),
                         preferred_element_type=jnp.float32)
    d = qq - 2.0 * d2 + xx
    iota_n = lax.broadcasted_iota(jnp.int32, (_MT, _N), 1)
    lane_k = lax.broadcasted_iota(jnp.int32, (_MT, _K), 1)
    acc = jnp.zeros((_MT, _K), jnp.int32)
    big = jnp.float32(3.0e38)
    for j in range(_K):
        r = jnp.min(d, axis=1, keepdims=True)
        cand = jnp.where(d == r, iota_n, _N)
        nidx = jnp.min(cand, axis=1, keepdims=True)
        acc = jnp.where(lane_k == j, nidx, acc)
        d = jnp.where(iota_n == nidx, big, d)
    out_ref[0] = acc


def _topk(q3, xt, x, interpret=False):
    grid = (_B, _M // _MT)
    return pl.pallas_call(
        _topk_body,
        grid=grid,
        in_specs=[
            pl.BlockSpec((1, _MT, _C), lambda b, m: (b, m, 0)),
            pl.BlockSpec((1, _N, _C), lambda b, m: (b, 0, 0)),
            pl.BlockSpec((1, _C, _N), lambda b, m: (b, 0, 0)),
        ],
        out_specs=pl.BlockSpec((1, _MT, _K), lambda b, m: (b, m, 0)),
        out_shape=jax.ShapeDtypeStruct((_B, _M, _K), jnp.int32),
        interpret=interpret,
    )(q3, xt, x)


_QT = 64

def _edge_body(neigh_ref, q_ref, w1_ref, b1_ref, w2_ref, b2_ref, out_ref):
    c = q_ref[...]
    n3 = neigh_ref[...]
    diff = (n3 - c[:, None, :]).reshape(_QT * _K, _C)
    w1a = w1_ref[:, :_C]
    w1b = w1_ref[:, _C:]
    dn = (((1,), (1,)), ((), ()))
    ha = lax.dot_general(diff, w1a, dn, preferred_element_type=jnp.float32)
    hb = lax.dot_general(c, w1b, dn, preferred_element_type=jnp.float32)
    h1 = ha.reshape(_QT, _K, 256) + hb[:, None, :] + b1_ref[...][None]
    h1 = jnp.where(h1 >= 0, h1, 0.2 * h1).reshape(_QT * _K, 256)
    h2 = lax.dot_general(h1, w2_ref[...], dn, preferred_element_type=jnp.float32)
    h2 = h2 + b2_ref[...]
    h2 = jnp.where(h2 >= 0, h2, 0.2 * h2)
    out_ref[...] = jnp.max(h2.reshape(_QT, _K, 512), axis=1)


def _edgeconv(neigh, qrows, W1, b1, W2, b2, interpret=False):
    grid = (_B * _M // _QT,)
    return pl.pallas_call(
        _edge_body,
        grid=grid,
        in_specs=[
            pl.BlockSpec((_QT, _K, _C), lambda t: (t, 0, 0)),
            pl.BlockSpec((_QT, _C), lambda t: (t, 0)),
            pl.BlockSpec((256, 256), lambda t: (0, 0)),
            pl.BlockSpec((1, 256), lambda t: (0, 0)),
            pl.BlockSpec((512, 256), lambda t: (0, 0)),
            pl.BlockSpec((1, 512), lambda t: (0, 0)),
        ],
        out_specs=pl.BlockSpec((_QT, 512), lambda t: (t, 0)),
        out_shape=jax.ShapeDtypeStruct((_B * _M, 512), jnp.float32),
        interpret=interpret,
    )(neigh, qrows, W1, b1, W2, b2)


def kernel(x, xyz, W1, b1, W2, b2):
    B, C, N = x.shape
    idx, sx, sy, sz = _fps(xyz[:, 0, :], xyz[:, 1, :], xyz[:, 2, :])
    samp = jnp.stack([sx, sy, sz], axis=1)

    xt_flat = jnp.transpose(x, (0, 2, 1)).reshape(B * N, C)
    offs = (jnp.arange(B, dtype=jnp.int32) * N)[:, None]
    gq = (idx + offs).reshape(B * _M)
    qrows = _sc_gather(xt_flat, gq, 256)

    knn = _topk(qrows.reshape(B, _M, C), xt_flat.reshape(B, N, C), x)
    gn = (knn + offs[:, :, None]).reshape(B * _M * _K)
    neigh = _sc_gather(xt_flat, gn, 256)

    out = _edgeconv(neigh.reshape(B * _M, _K, C), qrows,
                    W1, b1.reshape(1, -1), W2, b2.reshape(1, -1))
    x_processed = jnp.transpose(out.reshape(B, _M, 512), (0, 2, 1))
    return (x_processed, samp, idx)


# speedup vs baseline: 17.6227x; 1.0424x over previous
import functools

import jax
import jax.numpy as jnp
from jax import lax
from jax.experimental import pallas as pl
from jax.experimental.pallas import tpu as pltpu
from jax.experimental.pallas import tpu_sc as plsc

_B, _C, _N = 8, 128, 4096
_M = 1024
_K = 16


_FC = 1024
_NCH = _N // _FC
_FLUSH = 128


def _fps_body(xyzs_ref, idx_ref, sx_ref, sy_ref, sz_ref, dists_ref):
    iota3 = lax.broadcasted_iota(jnp.int32, (3 * _B, _FC), 1)
    iota1 = lax.broadcasted_iota(jnp.int32, (_B, _FC), 1)
    lane_f = lax.broadcasted_iota(jnp.int32, (_B, _FLUSH), 1)
    for c in range(_NCH):
        dists_ref[:, c * _FC:(c + 1) * _FC] = jnp.full((_B, _FC), 1e10,
                                                       dtype=jnp.float32)

    def step(i, far):
        far3 = jnp.concatenate([far, far, far], axis=0)
        csum = jnp.zeros((3 * _B, 1), jnp.float32)
        for c in range(_NCH):
            sl = pl.ds(c * _FC, _FC)
            xyz_c = xyzs_ref[:, sl]
            sel_c = iota3 == (far3 - c * _FC)
            csum = csum + jnp.sum(
                jnp.where(sel_c, xyz_c, jnp.float32(0.0)), axis=1, keepdims=True)
        cx = csum[0:_B]
        cy = csum[_B:2 * _B]
        cz = csum[2 * _B:3 * _B]
        best_v = jnp.full((_B, 1), -1.0, jnp.float32)
        best_i = jnp.zeros((_B, 1), jnp.int32)
        for c in range(_NCH):
            sl = pl.ds(c * _FC, _FC)
            dx = xyzs_ref[0:_B, sl] - cx
            dy = xyzs_ref[_B:2 * _B, sl] - cy
            dz = xyzs_ref[2 * _B:3 * _B, sl] - cz
            d = dx * dx + dy * dy + dz * dz
            dn = jnp.minimum(dists_ref[:, sl], d)
            dists_ref[:, sl] = dn
            mv = jnp.max(dn, axis=1, keepdims=True)
            mi = jnp.min(jnp.where(dn == mv, iota1, _FC),
                         axis=1, keepdims=True) + c * _FC
            upd = mv > best_v
            best_i = jnp.where(upd, mi, best_i)
            best_v = jnp.where(upd, mv, best_v)
        return best_i.astype(jnp.int32), cx, cy, cz

    def inner(i, carry):
        far, a_i, a_x, a_y, a_z = carry
        hit = lane_f == i
        a_i = jnp.where(hit, far, a_i)
        far, cx, cy, cz = step(i, far)
        a_x = jnp.where(hit, cx, a_x)
        a_y = jnp.where(hit, cy, a_y)
        a_z = jnp.where(hit, cz, a_z)
        return far, a_i, a_x, a_y, a_z

    def outer(o, far):
        zf = jnp.zeros((_B, _FLUSH), jnp.float32)
        zi = jnp.zeros((_B, _FLUSH), jnp.int32)
        far, a_i, a_x, a_y, a_z = lax.fori_loop(0, _FLUSH, inner,
                                                (far, zi, zf, zf, zf))
        sl = pl.ds(o * _FLUSH, _FLUSH)
        idx_ref[:, sl] = a_i
        sx_ref[:, sl] = a_x
        sy_ref[:, sl] = a_y
        sz_ref[:, sl] = a_z
        return far

    lax.fori_loop(0, _M // _FLUSH, outer, jnp.zeros((_B, 1), jnp.int32))


def _fps(xyzs, interpret=False):
    return pl.pallas_call(
        _fps_body,
        out_shape=[
            jax.ShapeDtypeStruct((_B, _M), jnp.int32),
            jax.ShapeDtypeStruct((_B, _M), jnp.float32),
            jax.ShapeDtypeStruct((_B, _M), jnp.float32),
            jax.ShapeDtypeStruct((_B, _M), jnp.float32),
        ],
        scratch_shapes=[pltpu.VMEM((_B, _N), jnp.float32)],
        interpret=interpret,
    )(xyzs)


def _sc_gather(table, gidx, rows_per_chunk):
    (R,) = gidx.shape
    V, D = table.shape
    info = plsc.get_sparse_core_info()
    nw = info.num_cores * info.num_subcores
    per_w = R // nw
    n_chunks = per_w // rows_per_chunk
    mesh = plsc.VectorSubcoreMesh(core_axis_name="c", subcore_axis_name="s")

    @functools.partial(
        pl.kernel,
        mesh=mesh,
        out_type=jax.ShapeDtypeStruct((R, D), jnp.float32),
        scratch_types=[
            pltpu.VMEM((rows_per_chunk,), jnp.int32),
            pltpu.VMEM((rows_per_chunk, D), jnp.float32),
            pltpu.SemaphoreType.DMA,
        ],
    )
    def gk(table_hbm, idx_hbm, out_hbm, idx_v, rows_v, sem):
        wid = lax.axis_index("s") * info.num_cores + lax.axis_index("c")

        def body(ci, carry):
            base = wid * per_w + ci * rows_per_chunk
            pltpu.sync_copy(idx_hbm.at[pl.ds(base, rows_per_chunk)], idx_v)
            pltpu.async_copy(table_hbm.at[idx_v], rows_v, sem).wait()
            pltpu.sync_copy(rows_v, out_hbm.at[pl.ds(base, rows_per_chunk)])
            return carry

        lax.fori_loop(0, n_chunks, body, 0)

    return gk(table, gidx)


_MT = 256


def _topk_body(q_ref, xt_ref, x_ref, out_ref):
    q = q_ref[0]
    xt = xt_ref[0]
    xsq = x_ref[0]
    qq = jnp.sum(q * q, axis=1, keepdims=True)
    xx = jnp.sum(xsq * xsq, axis=0, keepdims=True)
    d2 = lax.dot_general(q, xt, (((1,), (1,)), ((), ())),
                         preferred_element_type=jnp.float32)
    d = qq - 2.0 * d2 + xx
    iota_n = lax.broadcasted_iota(jnp.int32, (_MT, _N), 1)
    lane_k = lax.broadcasted_iota(jnp.int32, (_MT, _K), 1)
    acc = jnp.zeros((_MT, _K), jnp.int32)
    big = jnp.float32(3.0e38)
    for j in range(_K):
        nidx = jnp.argmin(d, axis=1)[:, None].astype(jnp.int32)
        acc = jnp.where(lane_k == j, nidx, acc)
        d = jnp.where(iota_n == nidx, big, d)
    out_ref[0] = acc


def _topk(q3, xt, x, interpret=False):
    grid = (_B, _M // _MT)
    return pl.pallas_call(
        _topk_body,
        grid=grid,
        in_specs=[
            pl.BlockSpec((1, _MT, _C), lambda b, m: (b, m, 0)),
            pl.BlockSpec((1, _N, _C), lambda b, m: (b, 0, 0)),
            pl.BlockSpec((1, _C, _N), lambda b, m: (b, 0, 0)),
        ],
        out_specs=pl.BlockSpec((1, _MT, _K), lambda b, m: (b, m, 0)),
        out_shape=jax.ShapeDtypeStruct((_B, _M, _K), jnp.int32),
        interpret=interpret,
    )(q3, xt, x)


_QT = 64

def _edge_body(neigh_ref, q_ref, w1_ref, b1_ref, w2_ref, b2_ref, out_ref):
    c = q_ref[...]
    n3 = neigh_ref[...]
    diff = (n3 - c[:, None, :]).reshape(_QT * _K, _C)
    w1a = w1_ref[:, :_C]
    w1b = w1_ref[:, _C:]
    dn = (((1,), (1,)), ((), ()))
    ha = lax.dot_general(diff, w1a, dn, preferred_element_type=jnp.float32)
    hb = lax.dot_general(c, w1b, dn, preferred_element_type=jnp.float32)
    hb = hb + b1_ref[...]
    h1 = ha.reshape(_QT, _K, 256) + hb[:, None, :]
    h1 = jnp.maximum(h1, 0.2 * h1).reshape(_QT * _K, 256)
    h2 = lax.dot_general(h1, w2_ref[...], dn, preferred_element_type=jnp.float32)
    h2 = h2 + b2_ref[...]
    h2 = jnp.maximum(h2, 0.2 * h2)
    h3 = h2.reshape(_QT, _K, 512)
    t = jnp.maximum(h3[:, :8], h3[:, 8:])
    t = jnp.maximum(t[:, :4], t[:, 4:])
    t = jnp.maximum(t[:, :2], t[:, 2:])
    out_ref[...] = jnp.maximum(t[:, 0], t[:, 1])


def _edgeconv(neigh, qrows, W1, b1, W2, b2, interpret=False):
    grid = (_B * _M // _QT,)
    return pl.pallas_call(
        _edge_body,
        grid=grid,
        in_specs=[
            pl.BlockSpec((_QT, _K, _C), lambda t: (t, 0, 0)),
            pl.BlockSpec((_QT, _C), lambda t: (t, 0)),
            pl.BlockSpec((256, 256), lambda t: (0, 0)),
            pl.BlockSpec((1, 256), lambda t: (0, 0)),
            pl.BlockSpec((512, 256), lambda t: (0, 0)),
            pl.BlockSpec((1, 512), lambda t: (0, 0)),
        ],
        out_specs=pl.BlockSpec((_QT, 512), lambda t: (t, 0)),
        out_shape=jax.ShapeDtypeStruct((_B * _M, 512), jnp.float32),
        interpret=interpret,
    )(neigh, qrows, W1, b1, W2, b2)


def kernel(x, xyz, W1, b1, W2, b2):
    B, C, N = x.shape
    idx, sx, sy, sz = _fps(jnp.transpose(xyz, (1, 0, 2)).reshape(3 * B, N))
    samp = jnp.stack([sx, sy, sz], axis=1)

    xt_flat = jnp.transpose(x, (0, 2, 1)).reshape(B * N, C)
    offs = (jnp.arange(B, dtype=jnp.int32) * N)[:, None]
    gq = (idx + offs).reshape(B * _M)
    qrows = _sc_gather(xt_flat, gq, 256)

    knn = _topk(qrows.reshape(B, _M, C), xt_flat.reshape(B, N, C), x)
    gn = (knn + offs[:, :, None]).reshape(B * _M * _K)
    neigh = _sc_gather(xt_flat, gn, 256)

    out = _edgeconv(neigh.reshape(B * _M, _K, C), qrows,
                    W1, b1.reshape(1, -1), W2, b2.reshape(1, -1))
    x_processed = jnp.transpose(out.reshape(B, _M, 512), (0, 2, 1))
    return (x_processed, samp, idx)
